# baseline (speedup 1.0000x reference)
import functools

import jax
import jax.numpy as jnp
from jax import lax
from jax.experimental import pallas as pl
from jax.experimental.pallas import tpu as pltpu
from jax.experimental.pallas import tpu_sc as plsc


def _build_glove(B, V, D):
    info = plsc.get_sparse_core_info()
    NC, NS, L = info.num_cores, info.num_subcores, info.num_lanes
    NW = NC * NS
    BPW = B // NW
    CH = 128
    NCH = BPW // CH
    RPC = 128 // D
    NGC = CH // L

    mesh = plsc.VectorSubcoreMesh(core_axis_name="c", subcore_axis_name="s")

    @functools.partial(
        pl.kernel,
        mesh=mesh,
        compiler_params=pltpu.CompilerParams(needs_layout_passes=False),
        out_type=jax.ShapeDtypeStruct((B,), jnp.float32),
        scratch_types=[
            pltpu.VMEM((NCH, CH), jnp.int32),
            pltpu.VMEM((NCH, CH), jnp.int32),
            pltpu.VMEM((NCH, CH), jnp.int32),
            pltpu.VMEM((NCH, CH), jnp.int32),
            pltpu.VMEM((2, CH, 128), jnp.float32),
            pltpu.VMEM((2, CH, 128), jnp.float32),
            pltpu.VMEM((BPW,), jnp.float32),
            pltpu.VMEM((BPW,), jnp.float32),
            pltpu.VMEM((BPW,), jnp.float32),
            pltpu.SemaphoreType.DMA,
            pltpu.SemaphoreType.DMA,
            pltpu.SemaphoreType.DMA,
        ],
    )
    def glove(ii_hbm, jj_hbm, wi_hbm, wj_hbm, bi_hbm, bj_hbm, out_hbm,
              raw_i, raw_j, cov_i, cov_j, buf_i, buf_j, bv_i, bv_j, out_v,
              sem0, sem1, semb):
        wid = lax.axis_index("s") * NC + lax.axis_index("c")
        base = wid * BPW
        sems = (sem0, sem1)

        for c in range(NCH):
            pltpu.sync_copy(ii_hbm.at[pl.ds(base + c * CH, CH)], raw_i.at[c])
            pltpu.sync_copy(jj_hbm.at[pl.ds(base + c * CH, CH)], raw_j.at[c])

        shift = RPC.bit_length() - 1
        for c in range(NCH):
            def cbody(k, carry, c=c):
                sl = pl.ds(k * L, L)
                cov_i[c, sl] = lax.shift_right_logical(raw_i[c, sl], shift)
                cov_j[c, sl] = lax.shift_right_logical(raw_j[c, sl], shift)
                return carry
            lax.fori_loop(0, NGC, cbody, 0)

        bias_copies = []
        for c in range(NCH):
            sl = pl.ds(c * CH, CH)
            bias_copies.append(
                pltpu.async_copy(bi_hbm.at[raw_i.at[c]], bv_i.at[sl], semb))
            bias_copies.append(
                pltpu.async_copy(bj_hbm.at[raw_j.at[c]], bv_j.at[sl], semb))

        def fire(c):
            slot = c % 2
            return (
                pltpu.async_copy(wi_hbm.at[cov_i.at[c]], buf_i.at[slot],
                                 sems[slot]),
                pltpu.async_copy(wj_hbm.at[cov_j.at[c]], buf_j.at[slot],
                                 sems[slot]),
            )

        lane = lax.iota(jnp.int32, L)
        row_copies = {0: fire(0)}

        for c in range(NCH):
            if c + 1 < NCH:
                row_copies[c + 1] = fire(c + 1)
            for cp in row_copies[c]:
                cp.wait()
            slot = c % 2
            bslot_i = buf_i.at[slot]
            bslot_j = buf_j.at[slot]

            def body(g, carry, c=c, bslot_i=bslot_i, bslot_j=bslot_j):
                sl = pl.ds(g * L, L)
                col_i = (raw_i[c, sl] & (RPC - 1)) * D
                col_j = (raw_j[c, sl] & (RPC - 1)) * D
                row_ids = g * L + lane
                acc = jnp.zeros((L,), jnp.float32)
                for d in range(D):
                    gi = plsc.load_gather(bslot_i, [row_ids, col_i + d])
                    gj = plsc.load_gather(bslot_j, [row_ids, col_j + d])
                    acc = acc + gi * gj
                out_v[pl.ds(c * CH + g * L, L)] = acc
                return carry

            lax.fori_loop(0, NGC, body, 0)

        for cp in bias_copies:
            cp.wait()

        def bias_body(k, carry):
            sl = pl.ds(k * L, L)
            out_v[sl] = out_v[sl] + bv_i[sl] + bv_j[sl]
            return carry

        lax.fori_loop(0, BPW // L, bias_body, 0)

        pltpu.sync_copy(out_v, out_hbm.at[pl.ds(base, BPW)])

    return glove


def kernel(i_indices, j_indices, wi, wj, bi, bj):
    B = i_indices.shape[0]
    V, D = wi.shape
    glove = _build_glove(B, V, D)
    return glove(i_indices, j_indices,
                 wi.reshape(V * D // 128, 128),
                 wj.reshape(V * D // 128, 128),
                 bi.reshape(V), bj.reshape(V))
